# baseline (speedup 1.0000x reference)
import numpy as np
import jax
import jax.numpy as jnp
from jax import lax
from jax.experimental import pallas as pl
from jax.experimental.pallas import tpu as pltpu
from jax.experimental.pallas import tpu_sc as plsc

_STRIDES = (8, 16, 32, 64, 128)
_HWS = ((64, 64), (32, 32), (16, 16), (8, 8), (4, 4))
_SAMPLE_RADIO_RATIO = 1.5
_P = sum(h * w for h, w in _HWS)
_PSC = 5504
_CH = 1376
_B = 8
_M = 64
_BIG = 99999999.0


def _point_xy() -> np.ndarray:
    xs, ys = [], []
    for (h, w), s in zip(_HWS, _STRIDES):
        gx = np.arange(w, dtype=np.float32) * s + s // 2
        gy = np.arange(h, dtype=np.float32) * s + s // 2
        yy, xx = np.meshgrid(gy, gx, indexing="ij")
        xs.append(xx.reshape(-1))
        ys.append(yy.reshape(-1))
    out = np.zeros((2, _PSC), np.float32)
    out[0, :_P] = np.concatenate(xs)
    out[1, :_P] = np.concatenate(ys)
    return out


_PTS_XY = _point_xy()
_BIG_ROW = np.full(_CH, _BIG, np.float32)


def _sc_body(gt_hbm, lb_hbm, ptx_hbm, pty_hbm, big_hbm,
             o_cls, o_cen, o_rl, o_rt, o_rr, o_rb,
             ba, bi, gb, lb, xyv, occ, oce, ocr):
    c = lax.axis_index("c")
    s = lax.axis_index("s")
    b = c * 4 + s % 4
    g = s // 4
    base = g * _CH
    lane = lax.broadcasted_iota(jnp.int32, (16,), 0)
    zf = jnp.zeros((16,), jnp.float32)
    zi = jnp.zeros((16,), jnp.int32)

    pltpu.sync_copy(gt_hbm.at[pl.ds(b * (_M * 4), _M * 4)], gb)
    pltpu.sync_copy(lb_hbm.at[pl.ds(b * _M, _M)], lb)
    pltpu.sync_copy(big_hbm, ba)
    pltpu.sync_copy(ptx_hbm.at[pl.ds(base, _CH)], xyv.at[pl.ds(0, _CH)])
    pltpu.sync_copy(pty_hbm.at[pl.ds(base, _CH)], xyv.at[pl.ds(_CH, _CH)])

    dx = lane & 3
    dy = lane >> 2

    def box_body(j, carry):
        col4 = (zi + j) * 4
        x1 = plsc.load_gather(gb, [col4])
        y1 = plsc.load_gather(gb, [col4 + 1])
        x2 = plsc.load_gather(gb, [col4 + 2])
        y2 = plsc.load_gather(gb, [col4 + 3])
        cx = (x1 + x2) / 2.0
        cy = (y1 + y2) / 2.0
        mf = zf + j.astype(jnp.float32)
        mh = jnp.maximum(x2 - x1, y2 - y1) * 0.5
        one = zi + 1
        lv_lo = (jnp.where(mh > 64.0, one, 0) + jnp.where(mh > 128.0, one, 0)
                 + jnp.where(mh > 256.0, one, 0)
                 + jnp.where(mh > 512.0, one, 0))

        def lv_body(it, carry2):
            lv = jnp.clip(lv_lo - 1 + it, 0, 4)
            w = 64 >> lv
            st = (8 << lv).astype(jnp.float32)
            off0 = (16384 - (16384 >> (2 * lv))) // 3
            lo = jnp.where(lv == 0, -1.0, (32 << lv).astype(jnp.float32))
            hi = jnp.where(lv == 4, 999999.0,
                           (64 << lv).astype(jnp.float32))
            bx = (cx / st).astype(jnp.int32) - 1
            by = (cy / st).astype(jnp.int32) - 1
            ix = bx + dx
            iy = by + dy
            inb = (ix >= 0) & (ix < w) & (iy >= 0) & (iy < w)
            x = ix.astype(jnp.float32) * st + st * 0.5
            y = iy.astype(jnp.float32) * st + st * 0.5
            l_ = x - x1
            t_ = y - y1
            r_ = x2 - x
            b_ = y2 - y
            omin = jnp.minimum(jnp.minimum(l_, t_), jnp.minimum(r_, b_))
            omax = jnp.maximum(jnp.maximum(l_, t_), jnp.maximum(r_, b_))
            area = (l_ + r_) * (t_ + b_)
            gmax = jnp.maximum(jnp.maximum(x - cx, cx - x),
                               jnp.maximum(y - cy, cy - y))
            pos = ((omin > 0.0) & (omax > lo) & (omax <= hi)
                   & (gmax < st * _SAMPLE_RADIO_RATIO) & inb)
            p = off0 + iy * w + ix - base
            pos = pos & (p >= 0) & (p < _CH)
            p = jnp.clip(p, 0, _CH - 1)
            cur = plsc.load_gather(ba, [p])
            better = pos & (area < cur)
            plsc.store_scatter(ba, [p], area, mask=better)
            plsc.store_scatter(bi, [p], mf, mask=better)
            return carry2
        lax.fori_loop(0, 3, lv_body, 0)
        return carry
    lax.fori_loop(0, _M, box_body, 0)

    def fin_body(i, carry):
        o16 = i * 16
        best_a = ba[pl.ds(o16, 16)]
        best_i = bi[pl.ds(o16, 16)]
        pos2 = best_a < _BIG
        idxv = jnp.clip(best_i.astype(jnp.int32), 0, _M - 1)
        i4 = idxv * 4
        x = xyv[pl.ds(o16, 16)]
        y = xyv[pl.ds(_CH + o16, 16)]
        x1 = plsc.load_gather(gb, [i4])
        y1 = plsc.load_gather(gb, [i4 + 1])
        x2 = plsc.load_gather(gb, [i4 + 2])
        y2 = plsc.load_gather(gb, [i4 + 3])
        lab = plsc.load_gather(lb, [idxv])
        l_ = x - x1
        t_ = y - y1
        r_ = x2 - x
        b_ = y2 - y
        lr_min = jnp.minimum(l_, r_)
        lr_max = jnp.maximum(l_, r_)
        tb_min = jnp.minimum(t_, b_)
        tb_max = jnp.maximum(t_, b_)
        val = lr_min * tb_min / (lr_max * tb_max + 1e-10)
        v = jnp.maximum(jnp.where(pos2, val, 1.0), 1e-30)
        ry = lax.bitcast_convert_type(
            0x5F3759DF - lax.shift_right_logical(
                lax.bitcast_convert_type(v, jnp.int32), 1), jnp.float32)
        for _ in range(2):
            ry = ry * (1.5 - 0.5 * v * ry * ry)
        sq = v * ry
        sq = 0.5 * (sq + v / sq)
        occ[pl.ds(o16, 16)] = jnp.where(pos2, lab, 0)
        oce[pl.ds(o16, 16)] = jnp.where(pos2, sq, -1.0)
        ocr[pl.ds(o16, 16)] = jnp.where(pos2, l_, -1.0)
        ocr[pl.ds(_CH + o16, 16)] = jnp.where(pos2, t_, -1.0)
        ocr[pl.ds(2 * _CH + o16, 16)] = jnp.where(pos2, r_, -1.0)
        ocr[pl.ds(3 * _CH + o16, 16)] = jnp.where(pos2, b_, -1.0)
        return carry
    lax.fori_loop(0, _CH // 16, fin_body, 0)

    obase = b * _PSC + base
    pltpu.sync_copy(occ, o_cls.at[pl.ds(obase, _CH)])
    pltpu.sync_copy(oce, o_cen.at[pl.ds(obase, _CH)])
    for r, oref in enumerate((o_rl, o_rt, o_rr, o_rb)):
        pltpu.sync_copy(ocr.at[pl.ds(r * _CH, _CH)], oref.at[pl.ds(obase, _CH)])


@jax.jit
def _gen_targets_sc(gt_box, labels):
    f32 = jnp.float32
    sc_fn = pl.kernel(
        _sc_body,
        mesh=plsc.VectorSubcoreMesh(core_axis_name="c", subcore_axis_name="s"),
        compiler_params=pltpu.CompilerParams(needs_layout_passes=False),
        out_type=(
            [jax.ShapeDtypeStruct((_B * _PSC,), jnp.int32)]
            + [jax.ShapeDtypeStruct((_B * _PSC,), f32) for _ in range(5)]),
        scratch_types=[
            pltpu.VMEM((_CH,), f32),
            pltpu.VMEM((_CH,), f32),
            pltpu.VMEM((_M * 4,), f32),
            pltpu.VMEM((_M,), jnp.int32),
            pltpu.VMEM((2 * _CH,), f32),
            pltpu.VMEM((_CH,), jnp.int32),
            pltpu.VMEM((_CH,), f32),
            pltpu.VMEM((4 * _CH,), f32),
        ],
    )
    outs = sc_fn(
        gt_box.astype(f32).reshape(_B * _M * 4),
        labels.astype(jnp.int32).reshape(_B * _M),
        jnp.asarray(_PTS_XY[0]), jnp.asarray(_PTS_XY[1]),
        jnp.asarray(_BIG_ROW))
    o_cls, o_cen, o_rl, o_rt, o_rr, o_rb = (
        o.reshape(_B, _PSC)[:, :_P] for o in outs)
    return (o_cls[..., None], o_cen[..., None],
            jnp.stack([o_rl, o_rt, o_rr, o_rb], axis=-1))


def kernel(cls_p3, cls_p4, cls_p5, cls_p6, cls_p7,
           cen_p3, cen_p4, cen_p5, cen_p6, cen_p7,
           reg_p3, reg_p4, reg_p5, reg_p6, reg_p7,
           gt_box, labels):
    return _gen_targets_sc(gt_box, labels)


# speedup vs baseline: 1.0888x; 1.0888x over previous
import functools

import numpy as np
import jax
import jax.numpy as jnp
from jax import lax
from jax.experimental import pallas as pl
from jax.experimental.pallas import tpu as pltpu
from jax.experimental.pallas import tpu_sc as plsc

_STRIDES = (8, 16, 32, 64, 128)
_LIMITS = ((-1.0, 64.0), (64.0, 128.0), (128.0, 256.0), (256.0, 512.0),
           (512.0, 999999.0))
_SAMPLE_RADIO_RATIO = 1.5
_HWS = ((64, 64), (32, 32), (16, 16), (8, 8), (4, 4))
_P = sum(h * w for h, w in _HWS)
_PSC = 5504
_CH = 1376
_BIG = 99999999.0
_LV = tuple(
    (sum(h * w for h, w in _HWS[:i]), _HWS[i][1], _HWS[i][0],
     float(_STRIDES[i]), _LIMITS[i][0], _LIMITS[i][1])
    for i in range(5))


def _point_xy() -> np.ndarray:
    xs, ys = [], []
    for (h, w), s in zip(_HWS, _STRIDES):
        gx = np.arange(w, dtype=np.float32) * s + s // 2
        gy = np.arange(h, dtype=np.float32) * s + s // 2
        yy, xx = np.meshgrid(gy, gx, indexing="ij")
        xs.append(xx.reshape(-1))
        ys.append(yy.reshape(-1))
    out = np.zeros((2, _PSC), np.float32)
    out[0, :_P] = np.concatenate(xs)
    out[1, :_P] = np.concatenate(ys)
    return out


_PTS_XY = _point_xy()


def _sc_body(gt_hbm, ptx_hbm, pty_hbm,
             o_cls, o_cen, o_rl, o_rt, o_rr, o_rb,
             ba, bi, gv, mb_a, mb_i, xyv, ocv, sh_a, sh_i):
    c = lax.axis_index("c")
    s = lax.axis_index("s")
    b = c * 4 + s % 4
    g = s // 4
    lane = lax.broadcasted_iota(jnp.int32, (16,), 0)
    zf = jnp.zeros((16,), jnp.float32)
    zi = jnp.zeros((16,), jnp.int32)

    pltpu.sync_copy(gt_hbm.at[pl.ds(b * 512, 512)], gv)

    def init_body(i, carry):
        ba[pl.ds(i * 16, 16)] = zf + _BIG
        bi[pl.ds(i * 16, 16)] = zf
        return carry
    lax.fori_loop(0, _PSC // 16, init_body, 0)

    dx = lane & 3
    dy = lane >> 2

    def box_body(j, carry):
        m = g * 16 + j
        col = zi + m
        x1 = plsc.load_gather(gv, [col])
        y1 = plsc.load_gather(gv, [col + 64])
        x2 = plsc.load_gather(gv, [col + 128])
        y2 = plsc.load_gather(gv, [col + 192])
        cx = (x1 + x2) / 2.0
        cy = (y1 + y2) / 2.0
        mf = zf + m.astype(jnp.float32)
        mh = jnp.maximum(x2 - x1, y2 - y1) * 0.5
        one = zi + 1
        lv_lo = (jnp.where(mh > 64.0, one, 0) + jnp.where(mh > 128.0, one, 0)
                 + jnp.where(mh > 256.0, one, 0)
                 + jnp.where(mh > 512.0, one, 0))

        def lv_body(it, carry2):
            lv = jnp.clip(lv_lo - 1 + it, 0, 4)
            w = 64 >> lv
            st = (8 << lv).astype(jnp.float32)
            off0 = (16384 - (16384 >> (2 * lv))) // 3
            lo = jnp.where(lv == 0, -1.0, (32 << lv).astype(jnp.float32))
            hi = jnp.where(lv == 4, 999999.0,
                           (64 << lv).astype(jnp.float32))
            bx = (cx / st).astype(jnp.int32) - 1
            by = (cy / st).astype(jnp.int32) - 1
            ix = bx + dx
            iy = by + dy
            inb = (ix >= 0) & (ix < w) & (iy >= 0) & (iy < w)
            x = ix.astype(jnp.float32) * st + st * 0.5
            y = iy.astype(jnp.float32) * st + st * 0.5
            l_ = x - x1
            t_ = y - y1
            r_ = x2 - x
            b_ = y2 - y
            omin = jnp.minimum(jnp.minimum(l_, t_), jnp.minimum(r_, b_))
            omax = jnp.maximum(jnp.maximum(l_, t_), jnp.maximum(r_, b_))
            area = (l_ + r_) * (t_ + b_)
            gmax = jnp.maximum(jnp.maximum(x - cx, cx - x),
                               jnp.maximum(y - cy, cy - y))
            pos = ((omin > 0.0) & (omax > lo) & (omax <= hi)
                   & (gmax < st * _SAMPLE_RADIO_RATIO) & inb)
            p = off0 + iy * w + ix
            p = jnp.clip(p, 0, _PSC - 1)
            cur = plsc.load_gather(ba, [p])
            better = pos & (area < cur)
            plsc.store_scatter(ba, [p], area, mask=better)
            plsc.store_scatter(bi, [p], mf, mask=better)
            return carry2
        lax.fori_loop(0, 3, lv_body, 0)
        return carry
    lax.fori_loop(0, 16, box_body, 0)

    pltpu.sync_copy(ba, sh_a.at[pl.ds(s * _PSC, _PSC)])
    pltpu.sync_copy(bi, sh_i.at[pl.ds(s * _PSC, _PSC)])
    plsc.subcore_barrier()

    base = g * _CH
    for gg in range(4):
        spub = gg * 4 + s % 4
        pltpu.sync_copy(sh_a.at[pl.ds(spub * _PSC + base, _CH)],
                        mb_a.at[pl.ds(gg * _CH, _CH)])
        pltpu.sync_copy(sh_i.at[pl.ds(spub * _PSC + base, _CH)],
                        mb_i.at[pl.ds(gg * _CH, _CH)])
    pltpu.sync_copy(ptx_hbm.at[pl.ds(base, _CH)], xyv.at[pl.ds(0, _CH)])
    pltpu.sync_copy(pty_hbm.at[pl.ds(base, _CH)], xyv.at[pl.ds(_CH, _CH)])

    def fin_body(i, carry):
        o16 = i * 16
        best_a = mb_a[pl.ds(o16, 16)]
        best_i = mb_i[pl.ds(o16, 16)]
        for gg in range(1, 4):
            ag = mb_a[pl.ds(gg * _CH + o16, 16)]
            take = ag < best_a
            best_a = jnp.where(take, ag, best_a)
            best_i = jnp.where(take, mb_i[pl.ds(gg * _CH + o16, 16)], best_i)
        pos2 = best_a < _BIG
        idxv = best_i.astype(jnp.int32)
        x = xyv[pl.ds(o16, 16)]
        y = xyv[pl.ds(_CH + o16, 16)]
        x1 = plsc.load_gather(gv, [idxv])
        y1 = plsc.load_gather(gv, [idxv + 64])
        x2 = plsc.load_gather(gv, [idxv + 128])
        y2 = plsc.load_gather(gv, [idxv + 192])
        lab = plsc.load_gather(gv, [idxv + 256])
        l_ = x - x1
        t_ = y - y1
        r_ = x2 - x
        b_ = y2 - y
        lr_min = jnp.minimum(l_, r_)
        lr_max = jnp.maximum(l_, r_)
        tb_min = jnp.minimum(t_, b_)
        tb_max = jnp.maximum(t_, b_)
        val = lr_min * tb_min / (lr_max * tb_max + 1e-10)
        v = jnp.maximum(jnp.where(pos2, val, 1.0), 1e-30)
        ry = lax.bitcast_convert_type(
            0x5F3759DF - lax.shift_right_logical(
                lax.bitcast_convert_type(v, jnp.int32), 1), jnp.float32)
        for _ in range(3):
            ry = ry * (1.5 - 0.5 * v * ry * ry)
        sq = v * ry
        sq = 0.5 * (sq + v / sq)
        ocv[pl.ds(o16, 16)] = jnp.where(pos2, lab, 0.0)
        ocv[pl.ds(_CH + o16, 16)] = jnp.where(pos2, sq, -1.0)
        ocv[pl.ds(2 * _CH + o16, 16)] = jnp.where(pos2, l_, -1.0)
        ocv[pl.ds(3 * _CH + o16, 16)] = jnp.where(pos2, t_, -1.0)
        ocv[pl.ds(4 * _CH + o16, 16)] = jnp.where(pos2, r_, -1.0)
        ocv[pl.ds(5 * _CH + o16, 16)] = jnp.where(pos2, b_, -1.0)
        return carry
    lax.fori_loop(0, _CH // 16, fin_body, 0)

    obase = b * _PSC + base
    for r, oref in enumerate((o_cls, o_cen, o_rl, o_rt, o_rr, o_rb)):
        pltpu.sync_copy(ocv.at[pl.ds(r * _CH, _CH)], oref.at[pl.ds(obase, _CH)])


@jax.jit
def _gen_targets_sc(gt_box, labels):
    bsz, m = labels.shape
    gtp = jnp.concatenate(
        [gt_box.astype(jnp.float32).transpose(0, 2, 1),
         labels.astype(jnp.float32)[:, None, :],
         jnp.zeros((bsz, 3, m), jnp.float32)],
        axis=1).reshape(bsz * 8 * m)
    pts = jnp.asarray(_PTS_XY)
    f32 = jnp.float32
    sc_fn = pl.kernel(
        _sc_body,
        mesh=plsc.VectorSubcoreMesh(core_axis_name="c", subcore_axis_name="s"),
        compiler_params=pltpu.CompilerParams(needs_layout_passes=False),
        out_type=[jax.ShapeDtypeStruct((bsz * _PSC,), f32) for _ in range(6)],
        scratch_types=[
            pltpu.VMEM((_PSC,), f32),
            pltpu.VMEM((_PSC,), f32),
            pltpu.VMEM((512,), f32),
            pltpu.VMEM((4 * _CH,), f32),
            pltpu.VMEM((4 * _CH,), f32),
            pltpu.VMEM((2 * _CH,), f32),
            pltpu.VMEM((6 * _CH,), f32),
            pltpu.VMEM_SHARED((16 * _PSC,), f32),
            pltpu.VMEM_SHARED((16 * _PSC,), f32),
        ],
    )
    outs = sc_fn(gtp, pts[0], pts[1])
    o_cls, o_cen, o_rl, o_rt, o_rr, o_rb = (
        o.reshape(bsz, _PSC)[:, :_P] for o in outs)
    cls_t = o_cls[..., None].astype(jnp.int32)
    cen_t = o_cen[..., None]
    reg_t = jnp.stack([o_rl, o_rt, o_rr, o_rb], axis=-1)
    return cls_t, cen_t, reg_t


def kernel(cls_p3, cls_p4, cls_p5, cls_p6, cls_p7,
           cen_p3, cen_p4, cen_p5, cen_p6, cen_p7,
           reg_p3, reg_p4, reg_p5, reg_p6, reg_p7,
           gt_box, labels):
    return _gen_targets_sc(gt_box, labels)


# speedup vs baseline: 1.1133x; 1.0226x over previous
import functools

import numpy as np
import jax
import jax.numpy as jnp
from jax import lax
from jax.experimental import pallas as pl
from jax.experimental.pallas import tpu as pltpu
from jax.experimental.pallas import tpu_sc as plsc

_STRIDES = (8, 16, 32, 64, 128)
_LIMITS = ((-1.0, 64.0), (64.0, 128.0), (128.0, 256.0), (256.0, 512.0),
           (512.0, 999999.0))
_SAMPLE_RADIO_RATIO = 1.5
_HWS = ((64, 64), (32, 32), (16, 16), (8, 8), (4, 4))
_P = sum(h * w for h, w in _HWS)
_PSC = 5504
_CH = 1376
_BIG = 99999999.0
_LV = tuple(
    (sum(h * w for h, w in _HWS[:i]), _HWS[i][1], _HWS[i][0],
     float(_STRIDES[i]), _LIMITS[i][0], _LIMITS[i][1])
    for i in range(5))


def _point_xy() -> np.ndarray:
    xs, ys = [], []
    for (h, w), s in zip(_HWS, _STRIDES):
        gx = np.arange(w, dtype=np.float32) * s + s // 2
        gy = np.arange(h, dtype=np.float32) * s + s // 2
        yy, xx = np.meshgrid(gy, gx, indexing="ij")
        xs.append(xx.reshape(-1))
        ys.append(yy.reshape(-1))
    out = np.zeros((2, _PSC), np.float32)
    out[0, :_P] = np.concatenate(xs)
    out[1, :_P] = np.concatenate(ys)
    return out


_PTS_XY = _point_xy()


def _sc_body(gt_hbm, ptx_hbm, pty_hbm,
             o_cls, o_cen, o_rl, o_rt, o_rr, o_rb,
             ba, bi, gv, mb_a, mb_i, xyv, ocv, sh_a, sh_i):
    c = lax.axis_index("c")
    s = lax.axis_index("s")
    b = c * 4 + s % 4
    g = s // 4
    lane = lax.broadcasted_iota(jnp.int32, (16,), 0)
    zf = jnp.zeros((16,), jnp.float32)
    zi = jnp.zeros((16,), jnp.int32)

    pltpu.sync_copy(gt_hbm.at[pl.ds(b * 512, 512)], gv)

    def init_body(i, carry):
        ba[pl.ds(i * 16, 16)] = zf + _BIG
        bi[pl.ds(i * 16, 16)] = zf
        return carry
    lax.fori_loop(0, _PSC // 16, init_body, 0)

    dx = lane & 3
    dy = lane >> 2

    def box_body(j, carry):
        m = g * 16 + j
        col = zi + m
        x1 = plsc.load_gather(gv, [col])
        y1 = plsc.load_gather(gv, [col + 64])
        x2 = plsc.load_gather(gv, [col + 128])
        y2 = plsc.load_gather(gv, [col + 192])
        cx = (x1 + x2) / 2.0
        cy = (y1 + y2) / 2.0
        mf = zf + m.astype(jnp.float32)
        for off0, w, h, st, lo, hi in _LV:
            bx = (cx * (1.0 / st)).astype(jnp.int32) - 1
            by = (cy * (1.0 / st)).astype(jnp.int32) - 1
            ix = bx + dx
            iy = by + dy
            inb = (ix >= 0) & (ix < w) & (iy >= 0) & (iy < h)
            x = ix.astype(jnp.float32) * st + float(int(st) // 2)
            y = iy.astype(jnp.float32) * st + float(int(st) // 2)
            l_ = x - x1
            t_ = y - y1
            r_ = x2 - x
            b_ = y2 - y
            omin = jnp.minimum(jnp.minimum(l_, t_), jnp.minimum(r_, b_))
            omax = jnp.maximum(jnp.maximum(l_, t_), jnp.maximum(r_, b_))
            area = (l_ + r_) * (t_ + b_)
            gmax = jnp.maximum(jnp.maximum(x - cx, cx - x),
                               jnp.maximum(y - cy, cy - y))
            pos = ((omin > 0.0) & (omax > lo) & (omax <= hi)
                   & (gmax < st * _SAMPLE_RADIO_RATIO) & inb)
            p = off0 + iy * w + ix
            p = jnp.clip(p, 0, _PSC - 1)
            cur = plsc.load_gather(ba, [p])
            better = pos & (area < cur)
            plsc.store_scatter(ba, [p], area, mask=better)
            plsc.store_scatter(bi, [p], mf, mask=better)
        return carry
    lax.fori_loop(0, 16, box_body, 0)

    pltpu.sync_copy(ba, sh_a.at[pl.ds(s * _PSC, _PSC)])
    pltpu.sync_copy(bi, sh_i.at[pl.ds(s * _PSC, _PSC)])
    plsc.subcore_barrier()

    base = g * _CH
    for gg in range(4):
        spub = gg * 4 + s % 4
        pltpu.sync_copy(sh_a.at[pl.ds(spub * _PSC + base, _CH)],
                        mb_a.at[pl.ds(gg * _CH, _CH)])
        pltpu.sync_copy(sh_i.at[pl.ds(spub * _PSC + base, _CH)],
                        mb_i.at[pl.ds(gg * _CH, _CH)])
    pltpu.sync_copy(ptx_hbm.at[pl.ds(base, _CH)], xyv.at[pl.ds(0, _CH)])
    pltpu.sync_copy(pty_hbm.at[pl.ds(base, _CH)], xyv.at[pl.ds(_CH, _CH)])

    def fin_body(i, carry):
        o16 = i * 16
        best_a = mb_a[pl.ds(o16, 16)]
        best_i = mb_i[pl.ds(o16, 16)]
        for gg in range(1, 4):
            ag = mb_a[pl.ds(gg * _CH + o16, 16)]
            take = ag < best_a
            best_a = jnp.where(take, ag, best_a)
            best_i = jnp.where(take, mb_i[pl.ds(gg * _CH + o16, 16)], best_i)
        pos2 = best_a < _BIG
        idxv = best_i.astype(jnp.int32)
        x = xyv[pl.ds(o16, 16)]
        y = xyv[pl.ds(_CH + o16, 16)]
        x1 = plsc.load_gather(gv, [idxv])
        y1 = plsc.load_gather(gv, [idxv + 64])
        x2 = plsc.load_gather(gv, [idxv + 128])
        y2 = plsc.load_gather(gv, [idxv + 192])
        lab = plsc.load_gather(gv, [idxv + 256])
        l_ = x - x1
        t_ = y - y1
        r_ = x2 - x
        b_ = y2 - y
        lr_min = jnp.minimum(l_, r_)
        lr_max = jnp.maximum(l_, r_)
        tb_min = jnp.minimum(t_, b_)
        tb_max = jnp.maximum(t_, b_)
        val = lr_min * tb_min / (lr_max * tb_max + 1e-10)
        v = jnp.maximum(jnp.where(pos2, val, 1.0), 1e-30)
        ry = lax.bitcast_convert_type(
            0x5F3759DF - lax.shift_right_logical(
                lax.bitcast_convert_type(v, jnp.int32), 1), jnp.float32)
        for _ in range(3):
            ry = ry * (1.5 - 0.5 * v * ry * ry)
        sq = v * ry
        sq = 0.5 * (sq + v / sq)
        ocv[pl.ds(o16, 16)] = jnp.where(pos2, lab, 0.0)
        ocv[pl.ds(_CH + o16, 16)] = jnp.where(pos2, sq, -1.0)
        ocv[pl.ds(2 * _CH + o16, 16)] = jnp.where(pos2, l_, -1.0)
        ocv[pl.ds(3 * _CH + o16, 16)] = jnp.where(pos2, t_, -1.0)
        ocv[pl.ds(4 * _CH + o16, 16)] = jnp.where(pos2, r_, -1.0)
        ocv[pl.ds(5 * _CH + o16, 16)] = jnp.where(pos2, b_, -1.0)
        return carry
    lax.fori_loop(0, _CH // 16, fin_body, 0)

    obase = b * _PSC + base
    for r, oref in enumerate((o_cls, o_cen, o_rl, o_rt, o_rr, o_rb)):
        pltpu.sync_copy(ocv.at[pl.ds(r * _CH, _CH)], oref.at[pl.ds(obase, _CH)])


@jax.jit
def _gen_targets_sc(gt_box, labels):
    bsz, m = labels.shape
    gtp = jnp.concatenate(
        [gt_box.astype(jnp.float32).transpose(0, 2, 1),
         labels.astype(jnp.float32)[:, None, :],
         jnp.zeros((bsz, 3, m), jnp.float32)],
        axis=1).reshape(bsz * 8 * m)
    pts = jnp.asarray(_PTS_XY)
    f32 = jnp.float32
    sc_fn = pl.kernel(
        _sc_body,
        mesh=plsc.VectorSubcoreMesh(core_axis_name="c", subcore_axis_name="s"),
        compiler_params=pltpu.CompilerParams(needs_layout_passes=False),
        out_type=[jax.ShapeDtypeStruct((bsz * _PSC,), f32) for _ in range(6)],
        scratch_types=[
            pltpu.VMEM((_PSC,), f32),
            pltpu.VMEM((_PSC,), f32),
            pltpu.VMEM((512,), f32),
            pltpu.VMEM((4 * _CH,), f32),
            pltpu.VMEM((4 * _CH,), f32),
            pltpu.VMEM((2 * _CH,), f32),
            pltpu.VMEM((6 * _CH,), f32),
            pltpu.VMEM_SHARED((16 * _PSC,), f32),
            pltpu.VMEM_SHARED((16 * _PSC,), f32),
        ],
    )
    outs = sc_fn(gtp, pts[0], pts[1])
    o_cls, o_cen, o_rl, o_rt, o_rr, o_rb = (
        o.reshape(bsz, _PSC)[:, :_P] for o in outs)
    cls_t = o_cls[..., None].astype(jnp.int32)
    cen_t = o_cen[..., None]
    reg_t = jnp.stack([o_rl, o_rt, o_rr, o_rb], axis=-1)
    return cls_t, cen_t, reg_t


def kernel(cls_p3, cls_p4, cls_p5, cls_p6, cls_p7,
           cen_p3, cen_p4, cen_p5, cen_p6, cen_p7,
           reg_p3, reg_p4, reg_p5, reg_p6, reg_p7,
           gt_box, labels):
    return _gen_targets_sc(gt_box, labels)


# speedup vs baseline: 1.1148x; 1.0013x over previous
import functools

import numpy as np
import jax
import jax.numpy as jnp
from jax import lax
from jax.experimental import pallas as pl
from jax.experimental.pallas import tpu as pltpu
from jax.experimental.pallas import tpu_sc as plsc

_STRIDES = (8, 16, 32, 64, 128)
_LIMITS = ((-1.0, 64.0), (64.0, 128.0), (128.0, 256.0), (256.0, 512.0),
           (512.0, 999999.0))
_SAMPLE_RADIO_RATIO = 1.5
_HWS = ((64, 64), (32, 32), (16, 16), (8, 8), (4, 4))
_P = sum(h * w for h, w in _HWS)
_PSC = 5504
_CH = 1376
_BIG = 99999999.0
_LV = tuple(
    (sum(h * w for h, w in _HWS[:i]), _HWS[i][1], _HWS[i][0],
     float(_STRIDES[i]), _LIMITS[i][0], _LIMITS[i][1])
    for i in range(5))


def _point_xy() -> np.ndarray:
    xs, ys = [], []
    for (h, w), s in zip(_HWS, _STRIDES):
        gx = np.arange(w, dtype=np.float32) * s + s // 2
        gy = np.arange(h, dtype=np.float32) * s + s // 2
        yy, xx = np.meshgrid(gy, gx, indexing="ij")
        xs.append(xx.reshape(-1))
        ys.append(yy.reshape(-1))
    out = np.zeros((2, _PSC), np.float32)
    out[0, :_P] = np.concatenate(xs)
    out[1, :_P] = np.concatenate(ys)
    return out


_PTS_XY = _point_xy()


def _sc_body(gt_hbm, ptx_hbm, pty_hbm,
             o_cls, o_cen, o_rl, o_rt, o_rr, o_rb,
             ba, bi, gv, mb_a, mb_i, xyv, ocv, sh_a, sh_i):
    c = lax.axis_index("c")
    s = lax.axis_index("s")
    b = c * 4 + s % 4
    g = s // 4
    lane = lax.broadcasted_iota(jnp.int32, (16,), 0)
    zf = jnp.zeros((16,), jnp.float32)
    zi = jnp.zeros((16,), jnp.int32)

    pltpu.sync_copy(gt_hbm.at[pl.ds(b * 512, 512)], gv)

    def init_body(i, carry):
        ba[pl.ds(i * 16, 16)] = zf + _BIG
        bi[pl.ds(i * 16, 16)] = zf
        return carry
    lax.fori_loop(0, _PSC // 16, init_body, 0)

    dx = lane & 3
    dy = lane >> 2

    def box_body(j, carry):
        m = g * 16 + j
        col = zi + m
        x1 = plsc.load_gather(gv, [col])
        y1 = plsc.load_gather(gv, [col + 64])
        x2 = plsc.load_gather(gv, [col + 128])
        y2 = plsc.load_gather(gv, [col + 192])
        cx = (x1 + x2) / 2.0
        cy = (y1 + y2) / 2.0
        mf = zf + m.astype(jnp.float32)
        for off0, w, h, st, lo, hi in _LV:
            bx = (cx * (1.0 / st)).astype(jnp.int32) - 1
            by = (cy * (1.0 / st)).astype(jnp.int32) - 1
            ix = bx + dx
            iy = by + dy
            inb = (ix >= 0) & (ix < w) & (iy >= 0) & (iy < h)
            x = ix.astype(jnp.float32) * st + float(int(st) // 2)
            y = iy.astype(jnp.float32) * st + float(int(st) // 2)
            l_ = x - x1
            t_ = y - y1
            r_ = x2 - x
            b_ = y2 - y
            omin = jnp.minimum(jnp.minimum(l_, t_), jnp.minimum(r_, b_))
            omax = jnp.maximum(jnp.maximum(l_, t_), jnp.maximum(r_, b_))
            area = (l_ + r_) * (t_ + b_)
            gmax = jnp.maximum(jnp.maximum(x - cx, cx - x),
                               jnp.maximum(y - cy, cy - y))
            pos = ((omin > 0.0) & (omax > lo) & (omax <= hi)
                   & (gmax < st * _SAMPLE_RADIO_RATIO) & inb)
            p = off0 + iy * w + ix
            p = jnp.clip(p, 0, _PSC - 1)
            cur = plsc.load_gather(ba, [p])
            better = pos & (area < cur)
            plsc.store_scatter(ba, [p], area, mask=better)
            plsc.store_scatter(bi, [p], mf, mask=better)
        return carry
    lax.fori_loop(0, 16, box_body, 0)

    base = g * _CH
    pltpu.sync_copy(ptx_hbm.at[pl.ds(base, _CH)], xyv.at[pl.ds(0, _CH)])
    pltpu.sync_copy(pty_hbm.at[pl.ds(base, _CH)], xyv.at[pl.ds(_CH, _CH)])
    pltpu.sync_copy(ba, sh_a.at[pl.ds(s * _PSC, _PSC)])
    pltpu.sync_copy(bi, sh_i.at[pl.ds(s * _PSC, _PSC)])
    plsc.subcore_barrier()

    for gg in range(4):
        spub = gg * 4 + s % 4
        pltpu.sync_copy(sh_a.at[pl.ds(spub * _PSC + base, _CH)],
                        mb_a.at[pl.ds(gg * _CH, _CH)])
        pltpu.sync_copy(sh_i.at[pl.ds(spub * _PSC + base, _CH)],
                        mb_i.at[pl.ds(gg * _CH, _CH)])

    def fin_body(i, carry):
        o16 = i * 16
        best_a = mb_a[pl.ds(o16, 16)]
        best_i = mb_i[pl.ds(o16, 16)]
        for gg in range(1, 4):
            ag = mb_a[pl.ds(gg * _CH + o16, 16)]
            take = ag < best_a
            best_a = jnp.where(take, ag, best_a)
            best_i = jnp.where(take, mb_i[pl.ds(gg * _CH + o16, 16)], best_i)
        pos2 = best_a < _BIG
        idxv = best_i.astype(jnp.int32)
        x = xyv[pl.ds(o16, 16)]
        y = xyv[pl.ds(_CH + o16, 16)]
        x1 = plsc.load_gather(gv, [idxv])
        y1 = plsc.load_gather(gv, [idxv + 64])
        x2 = plsc.load_gather(gv, [idxv + 128])
        y2 = plsc.load_gather(gv, [idxv + 192])
        lab = plsc.load_gather(gv, [idxv + 256])
        l_ = x - x1
        t_ = y - y1
        r_ = x2 - x
        b_ = y2 - y
        lr_min = jnp.minimum(l_, r_)
        lr_max = jnp.maximum(l_, r_)
        tb_min = jnp.minimum(t_, b_)
        tb_max = jnp.maximum(t_, b_)
        val = lr_min * tb_min / (lr_max * tb_max + 1e-10)
        v = jnp.maximum(jnp.where(pos2, val, 1.0), 1e-30)
        ry = lax.bitcast_convert_type(
            0x5F3759DF - lax.shift_right_logical(
                lax.bitcast_convert_type(v, jnp.int32), 1), jnp.float32)
        for _ in range(3):
            ry = ry * (1.5 - 0.5 * v * ry * ry)
        sq = v * ry
        sq = 0.5 * (sq + v / sq)
        ocv[pl.ds(o16, 16)] = jnp.where(pos2, lab, 0.0)
        ocv[pl.ds(_CH + o16, 16)] = jnp.where(pos2, sq, -1.0)
        ocv[pl.ds(2 * _CH + o16, 16)] = jnp.where(pos2, l_, -1.0)
        ocv[pl.ds(3 * _CH + o16, 16)] = jnp.where(pos2, t_, -1.0)
        ocv[pl.ds(4 * _CH + o16, 16)] = jnp.where(pos2, r_, -1.0)
        ocv[pl.ds(5 * _CH + o16, 16)] = jnp.where(pos2, b_, -1.0)
        return carry
    lax.fori_loop(0, _CH // 16, fin_body, 0)

    obase = b * _PSC + base
    for r, oref in enumerate((o_cls, o_cen, o_rl, o_rt, o_rr, o_rb)):
        pltpu.sync_copy(ocv.at[pl.ds(r * _CH, _CH)], oref.at[pl.ds(obase, _CH)])


@jax.jit
def _gen_targets_sc(gt_box, labels):
    bsz, m = labels.shape
    gtp = jnp.concatenate(
        [gt_box.astype(jnp.float32).transpose(0, 2, 1),
         labels.astype(jnp.float32)[:, None, :],
         jnp.zeros((bsz, 3, m), jnp.float32)],
        axis=1).reshape(bsz * 8 * m)
    pts = jnp.asarray(_PTS_XY)
    f32 = jnp.float32
    sc_fn = pl.kernel(
        _sc_body,
        mesh=plsc.VectorSubcoreMesh(core_axis_name="c", subcore_axis_name="s"),
        compiler_params=pltpu.CompilerParams(needs_layout_passes=False),
        out_type=[jax.ShapeDtypeStruct((bsz * _PSC,), f32) for _ in range(6)],
        scratch_types=[
            pltpu.VMEM((_PSC,), f32),
            pltpu.VMEM((_PSC,), f32),
            pltpu.VMEM((512,), f32),
            pltpu.VMEM((4 * _CH,), f32),
            pltpu.VMEM((4 * _CH,), f32),
            pltpu.VMEM((2 * _CH,), f32),
            pltpu.VMEM((6 * _CH,), f32),
            pltpu.VMEM_SHARED((16 * _PSC,), f32),
            pltpu.VMEM_SHARED((16 * _PSC,), f32),
        ],
    )
    outs = sc_fn(gtp, pts[0], pts[1])
    o_cls, o_cen, o_rl, o_rt, o_rr, o_rb = (
        o.reshape(bsz, _PSC)[:, :_P] for o in outs)
    cls_t = o_cls[..., None].astype(jnp.int32)
    cen_t = o_cen[..., None]
    reg_t = jnp.stack([o_rl, o_rt, o_rr, o_rb], axis=-1)
    return cls_t, cen_t, reg_t


def kernel(cls_p3, cls_p4, cls_p5, cls_p6, cls_p7,
           cen_p3, cen_p4, cen_p5, cen_p6, cen_p7,
           reg_p3, reg_p4, reg_p5, reg_p6, reg_p7,
           gt_box, labels):
    return _gen_targets_sc(gt_box, labels)
